# baseline (speedup 1.0000x reference)
import jax
import jax.numpy as jnp
from jax.experimental import pallas as pl


def _add_bias_kernel(x_ref, b_ref, o_ref):
    o_ref[...] = x_ref[...] + b_ref[...]


def _gat_conv_nobias(x, src, dst, W, a_src, a_dst, num_nodes):
    h = x @ W
    alpha_src = h @ a_src
    alpha_dst = h @ a_dst
    e = alpha_src[src] + alpha_dst[dst]
    e = jax.nn.leaky_relu(e, negative_slope=0.2)
    e_max = jax.ops.segment_max(e, dst, num_segments=num_nodes)
    e_max = jnp.where(jnp.isfinite(e_max), e_max, 0.0)
    e_exp = jnp.exp(e - e_max[dst])
    denom = jax.ops.segment_sum(e_exp, dst, num_segments=num_nodes)
    alpha = e_exp / (denom[dst] + 1e-16)
    out = jax.ops.segment_sum(h[src] * alpha[:, None], dst, num_segments=num_nodes)
    return out


def kernel(x, edge_index, W1, a_src1, a_dst1, b1, W2, a_src2, a_dst2, b2):
    n = x.shape[0]
    loops = jnp.arange(n, dtype=edge_index.dtype)
    src = jnp.concatenate([edge_index[0], loops])
    dst = jnp.concatenate([edge_index[1], loops])
    h = _gat_conv_nobias(x, src, dst, W1, a_src1, a_dst1, n)
    h = pl.pallas_call(
        _add_bias_kernel,
        out_shape=jax.ShapeDtypeStruct(h.shape, h.dtype),
    )(h, jnp.broadcast_to(b1, h.shape))
    h = jax.nn.relu(h)
    out = _gat_conv_nobias(h, src, dst, W2, a_src2, a_dst2, n)
    out = pl.pallas_call(
        _add_bias_kernel,
        out_shape=jax.ShapeDtypeStruct(out.shape, out.dtype),
    )(out, jnp.broadcast_to(b2, out.shape))
    return out


# speedup vs baseline: 24.0631x; 24.0631x over previous
import functools

import jax
import jax.numpy as jnp
from jax import lax
from jax.experimental import pallas as pl
from jax.experimental.pallas import tpu as pltpu
from jax.experimental.pallas import tpu_sc as plsc

N = 10000
NP = 10240
D = 128
H = 128
C = 40
CP = 128
ET = 330000
NC = 2
NS = 16
NW = NC * NS
K = 128
EPW = 10368
ETP = NW * EPW
NCH = EPW // K
BR = 1024
GRID = NP // BR


def _proj_body(x_ref, w_ref, asv_ref, adv_ref, h_ref, aso_ref, ado_ref):
    h = jnp.dot(x_ref[...], w_ref[...], preferred_element_type=jnp.float32)
    h_ref[...] = h
    aso_ref[...] = jnp.sum(h * asv_ref[...], axis=1)
    ado_ref[...] = jnp.sum(h * adv_ref[...], axis=1)


def _tc_transform(x, W, a_src, a_dst, F):
    return pl.pallas_call(
        _proj_body,
        grid=(GRID,),
        in_specs=[
            pl.BlockSpec((BR, x.shape[1]), lambda i: (i, 0)),
            pl.BlockSpec((x.shape[1], F), lambda i: (0, 0)),
            pl.BlockSpec((F,), lambda i: (0,)),
            pl.BlockSpec((F,), lambda i: (0,)),
        ],
        out_specs=[
            pl.BlockSpec((BR, F), lambda i: (i, 0)),
            pl.BlockSpec((BR,), lambda i: (i,)),
            pl.BlockSpec((BR,), lambda i: (i,)),
        ],
        out_shape=[
            jax.ShapeDtypeStruct((NP, F), jnp.float32),
            jax.ShapeDtypeStruct((NP,), jnp.float32),
            jax.ShapeDtypeStruct((NP,), jnp.float32),
        ],
    )(x, W, a_src, a_dst)


def _mid_body(a0_ref, a1_ref, d0_ref, d1_ref, b_ref, w_ref, asv_ref, adv_ref,
              h_ref, aso_ref, ado_ref):
    s = a0_ref[...] + a1_ref[...]
    d = d0_ref[...] + d1_ref[...]
    hin = s / (d[:, None] + 1e-16) + b_ref[...][None, :]
    hin = jnp.maximum(hin, 0.0)
    h2 = jnp.dot(hin, w_ref[...], preferred_element_type=jnp.float32)
    h_ref[...] = h2
    aso_ref[...] = jnp.sum(h2 * asv_ref[...], axis=1)
    ado_ref[...] = jnp.sum(h2 * adv_ref[...], axis=1)


def _tc_mid(acc, den, b1, W2, a_src2, a_dst2):
    return pl.pallas_call(
        _mid_body,
        grid=(GRID,),
        in_specs=[
            pl.BlockSpec((BR, H), lambda i: (i, 0)),
            pl.BlockSpec((BR, H), lambda i: (i + GRID, 0)),
            pl.BlockSpec((BR,), lambda i: (i,)),
            pl.BlockSpec((BR,), lambda i: (i + GRID,)),
            pl.BlockSpec((H,), lambda i: (0,)),
            pl.BlockSpec((H, CP), lambda i: (0, 0)),
            pl.BlockSpec((CP,), lambda i: (0,)),
            pl.BlockSpec((CP,), lambda i: (0,)),
        ],
        out_specs=[
            pl.BlockSpec((BR, CP), lambda i: (i, 0)),
            pl.BlockSpec((BR,), lambda i: (i,)),
            pl.BlockSpec((BR,), lambda i: (i,)),
        ],
        out_shape=[
            jax.ShapeDtypeStruct((NP, CP), jnp.float32),
            jax.ShapeDtypeStruct((NP,), jnp.float32),
            jax.ShapeDtypeStruct((NP,), jnp.float32),
        ],
    )(acc, acc, den, den, b1, W2, a_src2, a_dst2)


def _fin_body(a0_ref, a1_ref, d0_ref, d1_ref, b_ref, o_ref):
    s = a0_ref[...] + a1_ref[...]
    d = d0_ref[...] + d1_ref[...]
    o_ref[...] = s / (d[:, None] + 1e-16) + b_ref[...][None, :]


def _tc_final(acc, den, b2):
    return pl.pallas_call(
        _fin_body,
        grid=(GRID,),
        in_specs=[
            pl.BlockSpec((BR, CP), lambda i: (i, 0)),
            pl.BlockSpec((BR, CP), lambda i: (i + GRID, 0)),
            pl.BlockSpec((BR,), lambda i: (i,)),
            pl.BlockSpec((BR,), lambda i: (i + GRID,)),
            pl.BlockSpec((CP,), lambda i: (0,)),
        ],
        out_specs=pl.BlockSpec((BR, CP), lambda i: (i, 0)),
        out_shape=jax.ShapeDtypeStruct((NP, CP), jnp.float32),
    )(acc, acc, den, den, b2)


def _make_sc_edge(F):
    mesh = plsc.VectorSubcoreMesh(core_axis_name="c", subcore_axis_name="s")

    @functools.partial(
        pl.kernel,
        mesh=mesh,
        compiler_params=pltpu.CompilerParams(needs_layout_passes=False),
        out_type=[
            jax.ShapeDtypeStruct((NC * NP, F), jnp.float32),
            jax.ShapeDtypeStruct((NC * NP,), jnp.float32),
        ],
        scratch_types=[
            pltpu.VMEM((NP,), jnp.float32),
            pltpu.VMEM((NP,), jnp.float32),
            pltpu.VMEM((K,), jnp.int32),
            pltpu.VMEM((K,), jnp.int32),
            pltpu.VMEM((K,), jnp.float32),
            pltpu.VMEM((K, F), jnp.float32),
            pltpu.VMEM_SHARED((NP, F), jnp.float32),
            pltpu.VMEM_SHARED((NP,), jnp.float32),
            pltpu.SemaphoreType.DMA,
        ],
    )
    def sc_edge(src_hbm, dst_hbm, h_hbm, asrc_hbm, adst_hbm, zr_hbm, zv_hbm,
                acc_out, den_out,
                asrc_v, adst_v, srcv, dstv, pv, rows_v, acc_sh, den_sh, sem):
        cid = lax.axis_index("c")
        sid = lax.axis_index("s")
        wid = cid * NS + sid
        pltpu.sync_copy(asrc_hbm, asrc_v)
        pltpu.sync_copy(adst_hbm, adst_v)

        @pl.when(sid == 0)
        def _():
            pltpu.sync_copy(zr_hbm, acc_sh)
            pltpu.sync_copy(zv_hbm, den_sh)

        plsc.subcore_barrier()

        ebase = wid * EPW

        def chunk_body(ci, carry):
            base = ebase + ci * K
            pltpu.sync_copy(src_hbm.at[pl.ds(base, K)], srcv)
            pltpu.sync_copy(dst_hbm.at[pl.ds(base, K)], dstv)
            pltpu.async_copy(h_hbm.at[srcv], rows_v, sem).wait()

            def grp(j, c2):
                sidx = srcv[pl.ds(j * 16, 16)]
                didx = dstv[pl.ds(j * 16, 16)]
                av = plsc.load_gather(asrc_v, [sidx])
                bv = plsc.load_gather(adst_v, [didx])
                e = av + bv
                e = jnp.where(e >= 0.0, e, e * 0.2)
                p = jnp.exp(e)
                gidx = base + j * 16 + lax.iota(jnp.int32, 16)
                p = jnp.where(gidx < ET, p, 0.0)
                pv[pl.ds(j * 16, 16)] = p
                return c2

            lax.fori_loop(0, K // 16, grp, 0)
            pltpu.sync_copy(pv, den_sh.at[dstv], add=True)

            def rowgrp(j, c2):
                pvec = pv[pl.ds(j * 16, 16)]
                for k in range(16):
                    pk = pvec[k]
                    row = j * 16 + k
                    for r in range(F // 16):
                        sl = pl.ds(r * 16, 16)
                        rows_v[row, sl] = rows_v[row, sl] * pk
                return c2

            lax.fori_loop(0, K // 16, rowgrp, 0)
            pltpu.sync_copy(rows_v, acc_sh.at[dstv], add=True)
            return carry

        lax.fori_loop(0, NCH, chunk_body, 0)
        plsc.subcore_barrier()

        rpt = NP // NS
        rb = sid * rpt
        pltpu.sync_copy(acc_sh.at[pl.ds(rb, rpt)],
                        acc_out.at[pl.ds(cid * NP + rb, rpt)])
        pltpu.sync_copy(den_sh.at[pl.ds(rb, rpt)],
                        den_out.at[pl.ds(cid * NP + rb, rpt)])

    return sc_edge


_sc_edge_h = _make_sc_edge(H)
_sc_edge_c = _make_sc_edge(CP)


def kernel(x, edge_index, W1, a_src1, a_dst1, b1, W2, a_src2, a_dst2, b2):
    loops = jnp.arange(N, dtype=edge_index.dtype)
    src = jnp.pad(jnp.concatenate([edge_index[0], loops]), (0, ETP - ET))
    dst = jnp.pad(jnp.concatenate([edge_index[1], loops]), (0, ETP - ET))

    x_p = jnp.pad(x, ((0, NP - N), (0, 0)))
    W2p = jnp.pad(W2, ((0, 0), (0, CP - C)))
    a_src2p = jnp.pad(a_src2, (0, CP - C))
    a_dst2p = jnp.pad(a_dst2, (0, CP - C))
    b2p = jnp.pad(b2, (0, CP - C))

    zr_h = jnp.zeros((NP, H), jnp.float32)
    zr_c = jnp.zeros((NP, CP), jnp.float32)
    zv = jnp.zeros((NP,), jnp.float32)

    h1, as1, ad1 = _tc_transform(x_p, W1, a_src1, a_dst1, H)
    acc1, den1 = _sc_edge_h(src, dst, h1, as1, ad1, zr_h, zv)
    h2, as2, ad2 = _tc_mid(acc1, den1, b1, W2p, a_src2p, a_dst2p)
    acc2, den2 = _sc_edge_c(src, dst, h2, as2, ad2, zr_c, zv)
    out = _tc_final(acc2, den2, b2p)
    return out[:N, :C]
